# baseline (speedup 1.0000x reference)
import functools

import jax
import jax.numpy as jnp
from jax import lax
from jax.experimental import pallas as pl
from jax.experimental.pallas import tpu as pltpu
from jax.experimental.pallas import tpu_sc as plsc

D = 64
B, T, N = 8, 64, 1024

NC, NS = 2, 16
NW = NC * NS
NCHUNK = N // NW
NJ = D // 16


def _sc_base_kernel(regions_hbm, eids_hbm, bsp_hbm, slot_hbm, remb_hbm,
                    eemb_hbm, base_hbm, idx_v, reg_v, slot_v, out_v,
                    eids_v, eid_rows_v, bsp_v, cb_v, sem):
    c = lax.axis_index("c")
    s = lax.axis_index("s")
    wid = s * NC + c
    nbase = wid * NCHUNK

    pltpu.sync_copy(slot_hbm.at[pl.ds(nbase, NCHUNK)], slot_v)
    pltpu.sync_copy(eids_hbm, eids_v)
    pltpu.sync_copy(bsp_hbm, bsp_v)
    pltpu.async_copy(eemb_hbm.at[eids_v], eid_rows_v, sem).wait()

    for b in range(B):
        pltpu.sync_copy(regions_hbm.at[pl.ds(b * N + nbase, NCHUNK)], idx_v)
        pltpu.async_copy(remb_hbm.at[idx_v], reg_v, sem).wait()
        for j in range(NJ):
            sl = pl.ds(16 * j, 16)
            cb_v[sl] = eid_rows_v[b, sl] + bsp_v[sl]

        def body(n, carry):
            for j in range(NJ):
                sl = pl.ds(16 * j, 16)
                out_v[n, sl] = slot_v[n, sl] + reg_v[n, sl] + cb_v[sl]
            return carry

        lax.fori_loop(0, NCHUNK, body, 0)
        pltpu.sync_copy(out_v, base_hbm.at[pl.ds(b * N + nbase, NCHUNK)])


@functools.lru_cache(maxsize=1)
def _sc_base():
    return pl.kernel(
        _sc_base_kernel,
        out_type=jax.ShapeDtypeStruct((B * N, D), jnp.float32),
        mesh=plsc.VectorSubcoreMesh(core_axis_name="c", subcore_axis_name="s",
                                    num_cores=NC, num_subcores=NS),
        scratch_types=[
            pltpu.VMEM((NCHUNK,), jnp.int32),
            pltpu.VMEM((NCHUNK, D), jnp.float32),
            pltpu.VMEM((NCHUNK, D), jnp.float32),
            pltpu.VMEM((NCHUNK, D), jnp.float32),
            pltpu.VMEM((B,), jnp.int32),
            pltpu.VMEM((B, D), jnp.float32),
            pltpu.VMEM((D,), jnp.float32),
            pltpu.VMEM((D,), jnp.float32),
            pltpu.SemaphoreType.DMA,
        ],
        compiler_params=pltpu.CompilerParams(use_tc_tiling_on_sc=False),
    )


TT = 8


def _tc_body(s_ref, w_ref, base_ref, o_ref):
    s = s_ref[0]
    w = w_ref[0]
    base = base_ref[0]
    o_ref[0] = s[:, :, None] * w[None, None, :] + base[None, :, :]


def _tc_broadcast(spikes, wrow, base):
    return pl.pallas_call(
        _tc_body,
        grid=(B, T // TT),
        in_specs=[
            pl.BlockSpec((1, TT, N), lambda i, j: (i, j, 0)),
            pl.BlockSpec((1, D), lambda i, j: (0, 0)),
            pl.BlockSpec((1, N, D), lambda i, j: (i, 0, 0)),
        ],
        out_specs=pl.BlockSpec((1, TT, N, D), lambda i, j: (i, j, 0, 0)),
        out_shape=jax.ShapeDtypeStruct((B, T, N, D), jnp.float32),
    )(spikes, wrow, base)


def kernel(spikes, neuron_regions, eids, w_spike, b_spike, neuron_slot,
           region_emb, eid_emb):
    regions_flat = neuron_regions.astype(jnp.int32).reshape(B * N)
    base = _sc_base()(regions_flat, eids.astype(jnp.int32), b_spike,
                      neuron_slot, region_emb, eid_emb)
    wrow = w_spike.reshape(1, D)
    return _tc_broadcast(spikes, wrow, base.reshape(B, N, D))
